# baseline (speedup 1.0000x reference)
import functools

import jax
import jax.numpy as jnp
from jax.experimental import pallas as pl
from jax.experimental.pallas import tpu as pltpu
from jax.experimental.pallas import tpu_sc as plsc

H = 64


def _sc_gather_body(table_ref, idx_ref, out_ref, idx_v, tile_v, col_v, sem):
    c = jax.lax.axis_index("c")
    s = jax.lax.axis_index("s")

    @pl.when(jnp.logical_and(c == 0, s == 0))
    def _():
        pltpu.sync_copy(idx_ref, idx_v.at[pl.ds(0, 1)])
        i = idx_v[...][0]
        base = pl.multiple_of((i // 128) * 128, 128)
        pltpu.async_copy(table_ref.at[:, pl.ds(base, 128)], tile_v, sem).wait()
        c = i % 128
        ck = pl.multiple_of((c // 16) * 16, 16)
        cl = c % 16
        lane = jax.lax.iota(jnp.int32, 16)
        wl = [jnp.where(cl == lc, 1.0, 0.0) for lc in range(16)]
        for b in range(H // 16):
            out_chunk = jnp.zeros((16,), jnp.float32)
            for l in range(16):
                chunk = tile_v[16 * b + l, pl.ds(ck, 16)]
                v = chunk[0] * wl[0]
                for lc in range(1, 16):
                    v = v + chunk[lc] * wl[lc]
                out_chunk = jnp.where(lane == l, v, out_chunk)
            col_v[pl.ds(16 * b, 16)] = out_chunk
        pltpu.sync_copy(col_v, out_ref)


def _gather_row(table_t, idx):
    mesh = plsc.VectorSubcoreMesh(core_axis_name="c", subcore_axis_name="s")
    return pl.kernel(
        _sc_gather_body,
        out_type=jax.ShapeDtypeStruct((H,), jnp.float32),
        mesh=mesh,
        scratch_types=[
            pltpu.VMEM((16,), jnp.int32),
            pltpu.VMEM((H, 128), jnp.float32),
            pltpu.VMEM((H,), jnp.float32),
            pltpu.SemaphoreType.DMA,
        ],
    )(table_t, idx)


def _lstm_body(x_ref, h0_ref, c0_ref,
               wih0_ref, whh0_ref, b_ih0_ref, b_hh0_ref,
               wih1_ref, whh1_ref, b_ih1_ref, b_hh1_ref,
               out_ref, h_ref, c_ref):
    b0 = b_ih0_ref[...].reshape(1, 4 * H) + b_hh0_ref[...].reshape(1, 4 * H)
    b1 = b_ih1_ref[...].reshape(1, 4 * H) + b_hh1_ref[...].reshape(1, 4 * H)

    def gates_to_state(gates, cv):
        ig = jax.nn.sigmoid(gates[:, 0:H])
        fg = jax.nn.sigmoid(gates[:, H:2 * H])
        gg = jnp.tanh(gates[:, 2 * H:3 * H])
        og = jax.nn.sigmoid(gates[:, 3 * H:4 * H])
        c_new = fg * cv + ig * gg
        h_new = og * jnp.tanh(c_new)
        return h_new, c_new

    gates0 = (jnp.dot(x_ref[...].reshape(1, H), wih0_ref[...], preferred_element_type=jnp.float32)
              + jnp.dot(h0_ref[0], whh0_ref[...], preferred_element_type=jnp.float32)
              + b0)
    h1, c1 = gates_to_state(gates0, c0_ref[0])

    gates1 = (jnp.dot(h1, wih1_ref[...], preferred_element_type=jnp.float32)
              + jnp.dot(h0_ref[1], whh1_ref[...], preferred_element_type=jnp.float32)
              + b1)
    h2, c2 = gates_to_state(gates1, c0_ref[1])

    out_ref[0] = h2
    h_ref[0] = h1
    h_ref[1] = h2
    c_ref[0] = c1
    c_ref[1] = c2


def kernel(input, h0, c0, table, W_ih0, W_hh0, b_ih0, b_hh0, W_ih1, W_hh1, b_ih1, b_hh1):
    f32 = jnp.float32
    x = _gather_row(table.T, input)
    return tuple(pl.pallas_call(
        _lstm_body,
        out_shape=[
            jax.ShapeDtypeStruct((1, 1, H), f32),
            jax.ShapeDtypeStruct((2, 1, H), f32),
            jax.ShapeDtypeStruct((2, 1, H), f32),
        ],
    )(
        x, h0, c0,
        W_ih0.T, W_hh0.T, b_ih0, b_hh0,
        W_ih1.T, W_hh1.T, b_ih1, b_hh1,
    ))
